# baseline (speedup 1.0000x reference)
import functools

import jax
import jax.numpy as jnp
from jax import lax
from jax.experimental import pallas as pl
from jax.experimental.pallas import tpu as pltpu
from jax.experimental.pallas import tpu_sc as plsc

VOCAB = 1000
D = 512
_NROW = 4096
_L = 20
B_TOTAL = _NROW * _L


def _project_body(emb_ref, w_ref, b_ref, out_ref):
    p = lax.dot_general(
        emb_ref[...], w_ref[...],
        dimension_numbers=(((1,), (1,)), ((), ())),
        preferred_element_type=jnp.float32,
    )
    out_ref[...] = p + b_ref[...]


def _project_table(emb_table, W, b):
    return pl.pallas_call(
        _project_body,
        out_shape=jax.ShapeDtypeStruct((VOCAB, D), jnp.float32),
    )(emb_table, W, b.reshape(1, D))


_NW = 32
_B_PER_W = B_TOTAL // _NW
_CHUNK = 32
_NCHUNK = _B_PER_W // _CHUNK
_CPP = _NROW // _CHUNK
_NBUF = 6


def _make_gather():
    mesh = plsc.VectorSubcoreMesh(core_axis_name="c", subcore_axis_name="s")

    @functools.partial(
        pl.kernel,
        mesh=mesh,
        out_type=jax.ShapeDtypeStruct((_L, _NROW, D), jnp.float32),
        scratch_types=[
            pltpu.VMEM((_B_PER_W,), jnp.int32),
            pltpu.VMEM((_NBUF, _CHUNK, D), jnp.float32),
            pltpu.SemaphoreType.DMA,
            pltpu.SemaphoreType.DMA,
        ],
    )
    def gather_kernel(table_hbm, idx_hbm, out_hbm, idx_v, rows_v, gsem, wsem):
        wid = lax.axis_index("s") * 2 + lax.axis_index("c")
        base = wid * _B_PER_W
        kappa0 = wid * _NCHUNK
        pltpu.sync_copy(idx_hbm.at[pl.ds(base, _B_PER_W)], idx_v)

        def gcopy(g, slot):
            return pltpu.make_async_copy(
                table_hbm.at[idx_v.at[pl.ds(g * _CHUNK, _CHUNK)]],
                rows_v.at[slot], gsem)

        def wcopy(g, slot):
            kappa = kappa0 + g
            j = kappa // _CPP
            i0 = (kappa % _CPP) * _CHUNK
            return pltpu.make_async_copy(
                rows_v.at[slot],
                out_hbm.at[j, pl.ds(i0, _CHUNK)], wsem)

        for k in range(4):
            gcopy(k, k).start()

        for g in (0, 1):
            gcopy(g, g).wait()
            wcopy(g, g).start()
            gcopy(g + 4, g + 4).start()

        def body(g, _):
            slot = g % _NBUF
            gcopy(g, slot).wait()
            wcopy(g, slot).start()
            wcopy(g - 2, (g - 2) % _NBUF).wait()
            gcopy(g + 4, (g + 4) % _NBUF).start()
            return 0

        lax.fori_loop(2, _NCHUNK - 4, body, 0)

        def tail(g, _):
            gcopy(g, g % _NBUF).wait()
            wcopy(g, g % _NBUF).start()
            wcopy(g - 2, (g - 2) % _NBUF).wait()
            return 0

        lax.fori_loop(_NCHUNK - 4, _NCHUNK, tail, 0)
        wcopy(_NCHUNK - 2, (_NCHUNK - 2) % _NBUF).wait()
        wcopy(_NCHUNK - 1, (_NCHUNK - 1) % _NBUF).wait()

    return gather_kernel


def kernel(notation_ids, emb_table, W, b):
    P = _project_table(emb_table, W, b)
    ids_t = notation_ids.astype(jnp.int32).T.reshape(-1)
    out_planes = _make_gather()(P, ids_t)
    return out_planes.transpose(1, 0, 2)
